# baseline (speedup 1.0000x reference)
import functools

import numpy as np
import jax
import jax.numpy as jnp
from jax import lax
from jax.experimental import pallas as pl
from jax.experimental.pallas import tpu as pltpu

ALPHA_BASE = 0.3
MAX_DELTA = 0.5
EPS = 1e-12


def _fused_body(n_reg, logv, hs_ref, logits_ref, att_hbm, reg_ref, out_ref,
                att_vmem, inv_ref, sem):
    b = pl.program_id(0)

    @pl.when(b == 0)
    def _load_attractors():
        cp = pltpu.make_async_copy(att_hbm, att_vmem, sem)
        cp.start()
        cp.wait()
        KC = 512
        def _norm_step(i, _):
            blk = att_vmem[pl.ds(i * KC, KC), :]
            sq = jnp.sum(blk * blk, axis=1)
            inv_ref[0, pl.ds(i * KC, KC)] = (
                1.0 / jnp.maximum(jnp.sqrt(sq), EPS))
            return 0
        lax.fori_loop(0, att_vmem.shape[0] // KC, _norm_step, 0)

    Bb, S, D = hs_ref.shape
    SC = 32
    def _mean_step(i, acc):
        return acc + jnp.sum(hs_ref[:, pl.ds(i * SC, SC), :], axis=1)
    v_sum = lax.fori_loop(0, S // SC, _mean_step,
                          jnp.zeros((Bb, D), jnp.float32))
    v_raw = v_sum / S
    vn = jnp.sqrt(jnp.sum(v_raw * v_raw, axis=1, keepdims=True))
    v_unit = v_raw / jnp.maximum(vn, EPS)

    att = att_vmem[...]
    inv = inv_ref[0, :]
    sims = lax.dot_general(v_unit, att, (((1,), (1,)), ((), ())),
                           preferred_element_type=jnp.float32)
    sims = sims * inv[None, :]
    best = jnp.max(sims, axis=1)
    idx = jnp.argmax(sims, axis=1)
    iota_k = lax.broadcasted_iota(jnp.int32, sims.shape, 1)
    onehot = (iota_k == idx[:, None]).astype(jnp.float32)
    closest = lax.dot_general(onehot, att, (((1,), (0,)), ((), ())),
                              preferred_element_type=jnp.float32)
    cn = jnp.sqrt(jnp.sum(closest * closest, axis=1, keepdims=True))
    closest_n = closest / jnp.maximum(cn, EPS)

    V = logits_ref.shape[1]
    VC = 3200
    def _max_step(i, m):
        return jnp.maximum(m, jnp.max(logits_ref[:, pl.ds(i * VC, VC)],
                                      axis=1, keepdims=True))
    m = lax.fori_loop(0, V // VC, _max_step,
                      jnp.full((Bb, 1), -jnp.inf, jnp.float32))
    def _sum_step(i, carry):
        z, s1 = carry
        xc = logits_ref[:, pl.ds(i * VC, VC)] - m
        ec = jnp.exp(xc)
        return z + jnp.sum(ec, axis=1), s1 + jnp.sum(xc * ec, axis=1)
    z, s1 = lax.fori_loop(0, V // VC, _sum_step,
                          (jnp.zeros((Bb,), jnp.float32),
                           jnp.zeros((Bb,), jnp.float32)))
    entropy = jnp.log(z) - s1 / z
    conf = 1.0 - entropy / logv

    reg = reg_ref[...]
    rn = jnp.sqrt(jnp.sum(reg * reg, axis=1, keepdims=True))
    reg_n = reg / jnp.maximum(rn, EPS)
    rsims = lax.dot_general(v_unit, reg_n, (((1,), (1,)), ((), ())),
                            preferred_element_type=jnp.float32)
    iota_r = lax.broadcasted_iota(jnp.int32, rsims.shape, 1)
    rsims = jnp.where(iota_r < n_reg, rsims, -1e30)
    penalty = jnp.max(rsims, axis=1)

    alpha = (ALPHA_BASE * (1.0 - best)) * conf * (1.0 - penalty * 0.5)
    delta = jnp.clip(closest_n - v_unit, -MAX_DELTA, MAX_DELTA)
    steering = alpha[:, None] * delta

    def _add_step(i, _):
        out_ref[:, pl.ds(i * SC, SC), :] = (
            hs_ref[:, pl.ds(i * SC, SC), :] + steering[:, None, :])
        return 0
    lax.fori_loop(0, S // SC, _add_step, 0)


def kernel(hidden_states, logits, attractors, uncertainty_regions):
    B, S, D = hidden_states.shape
    K = attractors.shape[0]
    V = logits.shape[-1]
    R = uncertainty_regions.shape[0]
    logits2d = logits[:, -1, :]
    Rp = max(8, ((R + 7) // 8) * 8)
    reg_p = jnp.pad(uncertainty_regions, ((0, Rp - R), (0, 0)))

    Bb = 8
    body = functools.partial(_fused_body, R, float(np.log(V)))
    return pl.pallas_call(
        body,
        grid=(B // Bb,),
        in_specs=[
            pl.BlockSpec((Bb, S, D), lambda b: (b, 0, 0)),
            pl.BlockSpec((Bb, V), lambda b: (b, 0)),
            pl.BlockSpec(memory_space=pl.ANY),
            pl.BlockSpec((Rp, D), lambda b: (0, 0)),
        ],
        out_specs=pl.BlockSpec((Bb, S, D), lambda b: (b, 0, 0)),
        out_shape=jax.ShapeDtypeStruct((B, S, D), jnp.float32),
        scratch_shapes=[
            pltpu.VMEM((K, D), jnp.float32),
            pltpu.VMEM((1, K), jnp.float32),
            pltpu.SemaphoreType.DMA,
        ],
        compiler_params=pltpu.CompilerParams(
            vmem_limit_bytes=67108864),
    )(hidden_states, logits2d, attractors, reg_p)


# speedup vs baseline: 1.4352x; 1.4352x over previous
import functools

import numpy as np
import jax
import jax.numpy as jnp
from jax import lax
from jax.experimental import pallas as pl
from jax.experimental.pallas import tpu as pltpu

ALPHA_BASE = 0.3
MAX_DELTA = 0.5
EPS = 1e-12


def _fused_body(n_reg, logv, hs_ref, logits_ref, att_hbm, reg_ref, out_ref,
                att_vmem, inv_ref, sem):
    b = pl.program_id(0)

    @pl.when(b == 0)
    def _load_attractors():
        cp = pltpu.make_async_copy(att_hbm, att_vmem, sem)
        cp.start()
        cp.wait()
        KC = 512
        def _norm_step(i, _):
            blk = att_vmem[pl.ds(i * KC, KC), :]
            sq = jnp.sum(blk * blk, axis=1)
            inv_ref[0, pl.ds(i * KC, KC)] = (
                1.0 / jnp.maximum(jnp.sqrt(sq), EPS))
            return 0
        lax.fori_loop(0, att_vmem.shape[0] // KC, _norm_step, 0)

    Bb, S, D = hs_ref.shape
    SC = 32
    def _mean_step(i, acc):
        return acc + jnp.sum(hs_ref[:, pl.ds(i * SC, SC), :], axis=1)
    v_sum = lax.fori_loop(0, S // SC, _mean_step,
                          jnp.zeros((Bb, D), jnp.float32))
    v_raw = v_sum / S
    vn = jnp.sqrt(jnp.sum(v_raw * v_raw, axis=1, keepdims=True))
    v_unit = v_raw / jnp.maximum(vn, EPS)

    K = att_vmem.shape[0]
    KB = 1024
    best = jnp.full((Bb,), -jnp.inf, jnp.float32)
    idx = jnp.zeros((Bb,), jnp.int32)
    for k in range(K // KB):
        att_c = att_vmem[k * KB:(k + 1) * KB, :]
        s_c = lax.dot_general(v_unit, att_c, (((1,), (1,)), ((), ())),
                              preferred_element_type=jnp.float32)
        s_c = s_c * inv_ref[0, k * KB:(k + 1) * KB][None, :]
        m_c = jnp.max(s_c, axis=1)
        i_c = jnp.argmax(s_c, axis=1).astype(jnp.int32) + k * KB
        upd = m_c > best
        idx = jnp.where(upd, i_c, idx)
        best = jnp.where(upd, m_c, best)

    closest = jnp.zeros((Bb, D), jnp.float32)
    for k in range(K // KB):
        att_c = att_vmem[k * KB:(k + 1) * KB, :]
        iota_k = lax.broadcasted_iota(jnp.int32, (Bb, KB), 1) + k * KB
        oh = (iota_k == idx[:, None]).astype(jnp.float32)
        closest = closest + lax.dot_general(
            oh, att_c, (((1,), (0,)), ((), ())),
            preferred_element_type=jnp.float32)
    cn = jnp.sqrt(jnp.sum(closest * closest, axis=1, keepdims=True))
    closest_n = closest / jnp.maximum(cn, EPS)

    nch = logits_ref.shape[0] // Bb
    lane = lax.broadcasted_iota(jnp.int32, (Bb,), 0)
    conf = jnp.zeros((Bb,), jnp.float32)
    for bi in range(Bb):
        x2 = logits_ref[bi * nch:(bi + 1) * nch, :]
        m_b = jnp.max(x2)
        xc = x2 - m_b
        ec = jnp.exp(xc)
        z_b = jnp.sum(ec)
        s1_b = jnp.sum(xc * ec)
        conf_b = 1.0 - (jnp.log(z_b) - s1_b / z_b) / logv
        conf = jnp.where(lane == bi, conf_b, conf)

    reg = reg_ref[...]
    rn = jnp.sqrt(jnp.sum(reg * reg, axis=1, keepdims=True))
    reg_n = reg / jnp.maximum(rn, EPS)
    rsims = lax.dot_general(v_unit, reg_n, (((1,), (1,)), ((), ())),
                            preferred_element_type=jnp.float32)
    iota_r = lax.broadcasted_iota(jnp.int32, rsims.shape, 1)
    rsims = jnp.where(iota_r < n_reg, rsims, -1e30)
    penalty = jnp.max(rsims, axis=1)

    alpha = (ALPHA_BASE * (1.0 - best)) * conf * (1.0 - penalty * 0.5)
    delta = jnp.clip(closest_n - v_unit, -MAX_DELTA, MAX_DELTA)
    steering = alpha[:, None] * delta

    def _add_step(i, _):
        out_ref[:, pl.ds(i * SC, SC), :] = (
            hs_ref[:, pl.ds(i * SC, SC), :] + steering[:, None, :])
        return 0
    lax.fori_loop(0, S // SC, _add_step, 0)


def kernel(hidden_states, logits, attractors, uncertainty_regions):
    B, S, D = hidden_states.shape
    K = attractors.shape[0]
    V = logits.shape[-1]
    R = uncertainty_regions.shape[0]
    logits_flat = logits[:, -1, :].reshape(B * V // 128, 128)
    Rp = max(8, ((R + 7) // 8) * 8)
    reg_p = jnp.pad(uncertainty_regions, ((0, Rp - R), (0, 0)))

    Bb = 8
    body = functools.partial(_fused_body, R, float(np.log(V)))
    return pl.pallas_call(
        body,
        grid=(B // Bb,),
        in_specs=[
            pl.BlockSpec((Bb, S, D), lambda b: (b, 0, 0)),
            pl.BlockSpec((Bb * V // 128, 128), lambda b: (b, 0)),
            pl.BlockSpec(memory_space=pl.ANY),
            pl.BlockSpec((Rp, D), lambda b: (0, 0)),
        ],
        out_specs=pl.BlockSpec((Bb, S, D), lambda b: (b, 0, 0)),
        out_shape=jax.ShapeDtypeStruct((B, S, D), jnp.float32),
        scratch_shapes=[
            pltpu.VMEM((K, D), jnp.float32),
            pltpu.VMEM((1, K), jnp.float32),
            pltpu.SemaphoreType.DMA,
        ],
        compiler_params=pltpu.CompilerParams(
            vmem_limit_bytes=67108864),
    )(hidden_states, logits_flat, attractors, reg_p)


# speedup vs baseline: 1.6322x; 1.1373x over previous
import functools

import numpy as np
import jax
import jax.numpy as jnp
from jax import lax
from jax.experimental import pallas as pl
from jax.experimental.pallas import tpu as pltpu

ALPHA_BASE = 0.3
MAX_DELTA = 0.5
EPS = 1e-12


def _fused_body(n_reg, logv, hs_ref, logits_ref, att_hbm, reg_ref, out_ref,
                att_vmem, inv_ref, sem):
    b = pl.program_id(0)

    @pl.when(b == 0)
    def _load_attractors():
        cp = pltpu.make_async_copy(att_hbm, att_vmem, sem)
        cp.start()
        cp.wait()
        KC = 512
        def _norm_step(i, _):
            blk = att_vmem[pl.ds(i * KC, KC), :]
            sq = jnp.sum(blk * blk, axis=1)
            inv_ref[0, pl.ds(i * KC, KC)] = (
                1.0 / jnp.maximum(jnp.sqrt(sq), EPS))
            return 0
        lax.fori_loop(0, att_vmem.shape[0] // KC, _norm_step, 0)

    Bb, S, D = hs_ref.shape
    SC = 32
    def _mean_step(i, acc):
        return acc + jnp.sum(hs_ref[:, pl.ds(i * SC, SC), :], axis=1)
    v_sum = lax.fori_loop(0, S // SC, _mean_step,
                          jnp.zeros((Bb, D), jnp.float32))
    v_raw = v_sum / S
    vn = jnp.sqrt(jnp.sum(v_raw * v_raw, axis=1, keepdims=True))
    v_unit = v_raw / jnp.maximum(vn, EPS)

    K = att_vmem.shape[0]
    KB = 1024
    best = jnp.full((Bb,), -jnp.inf, jnp.float32)
    idx = jnp.zeros((Bb,), jnp.int32)
    for k in range(K // KB):
        att_c = att_vmem[k * KB:(k + 1) * KB, :]
        s_c = lax.dot_general(v_unit, att_c, (((1,), (1,)), ((), ())),
                              preferred_element_type=jnp.float32)
        s_c = s_c * inv_ref[0, k * KB:(k + 1) * KB][None, :]
        m_c = jnp.max(s_c, axis=1)
        i_c = jnp.argmax(s_c, axis=1).astype(jnp.int32) + k * KB
        upd = m_c > best
        idx = jnp.where(upd, i_c, idx)
        best = jnp.where(upd, m_c, best)

    lane = lax.broadcasted_iota(jnp.int32, (Bb,), 0)
    closest = jnp.zeros((Bb, D), jnp.float32)
    for bi in range(Bb):
        idx_i = jnp.sum(jnp.where(lane == bi, idx, 0))
        row = att_vmem[pl.ds(idx_i, 1), :]
        maskf = jnp.where(lane == bi, 1.0, 0.0)
        closest = closest + maskf[:, None] * row
    cn = jnp.sqrt(jnp.sum(closest * closest, axis=1, keepdims=True))
    closest_n = closest / jnp.maximum(cn, EPS)

    nch = logits_ref.shape[0] // Bb
    conf = jnp.zeros((Bb,), jnp.float32)
    for bi in range(Bb):
        x2 = logits_ref[bi * nch:(bi + 1) * nch, :]
        m_b = jnp.max(x2)
        xc = x2 - m_b
        ec = jnp.exp(xc)
        z_b = jnp.sum(ec)
        s1_b = jnp.sum(xc * ec)
        conf_b = 1.0 - (jnp.log(z_b) - s1_b / z_b) / logv
        conf = jnp.where(lane == bi, conf_b, conf)

    reg = reg_ref[...]
    rn = jnp.sqrt(jnp.sum(reg * reg, axis=1, keepdims=True))
    reg_n = reg / jnp.maximum(rn, EPS)
    rsims = lax.dot_general(v_unit, reg_n, (((1,), (1,)), ((), ())),
                            preferred_element_type=jnp.float32)
    iota_r = lax.broadcasted_iota(jnp.int32, rsims.shape, 1)
    rsims = jnp.where(iota_r < n_reg, rsims, -1e30)
    penalty = jnp.max(rsims, axis=1)

    alpha = (ALPHA_BASE * (1.0 - best)) * conf * (1.0 - penalty * 0.5)
    delta = jnp.clip(closest_n - v_unit, -MAX_DELTA, MAX_DELTA)
    steering = alpha[:, None] * delta

    def _add_step(i, _):
        out_ref[:, pl.ds(i * SC, SC), :] = (
            hs_ref[:, pl.ds(i * SC, SC), :] + steering[:, None, :])
        return 0
    lax.fori_loop(0, S // SC, _add_step, 0)


def kernel(hidden_states, logits, attractors, uncertainty_regions):
    B, S, D = hidden_states.shape
    K = attractors.shape[0]
    V = logits.shape[-1]
    R = uncertainty_regions.shape[0]
    logits_flat = logits[:, -1, :].reshape(B * V // 128, 128)
    Rp = max(8, ((R + 7) // 8) * 8)
    reg_p = jnp.pad(uncertainty_regions, ((0, Rp - R), (0, 0)))

    Bb = 8
    body = functools.partial(_fused_body, R, float(np.log(V)))
    return pl.pallas_call(
        body,
        grid=(B // Bb,),
        in_specs=[
            pl.BlockSpec((Bb, S, D), lambda b: (b, 0, 0)),
            pl.BlockSpec((Bb * V // 128, 128), lambda b: (b, 0)),
            pl.BlockSpec(memory_space=pl.ANY),
            pl.BlockSpec((Rp, D), lambda b: (0, 0)),
        ],
        out_specs=pl.BlockSpec((Bb, S, D), lambda b: (b, 0, 0)),
        out_shape=jax.ShapeDtypeStruct((B, S, D), jnp.float32),
        scratch_shapes=[
            pltpu.VMEM((K, D), jnp.float32),
            pltpu.VMEM((1, K), jnp.float32),
            pltpu.SemaphoreType.DMA,
        ],
        compiler_params=pltpu.CompilerParams(
            vmem_limit_bytes=67108864),
    )(hidden_states, logits_flat, attractors, reg_p)


# speedup vs baseline: 1.9727x; 1.2086x over previous
import functools

import numpy as np
import jax
import jax.numpy as jnp
from jax import lax
from jax.experimental import pallas as pl
from jax.experimental.pallas import tpu as pltpu

ALPHA_BASE = 0.3
MAX_DELTA = 0.5
EPS = 1e-12


def _fused_body(n_reg, logv, hs_ref, logits_ref, att_hbm, reg_ref, out_ref,
                att_vmem, inv_ref, rows_ref, sem):
    b = pl.program_id(0)

    @pl.when(b == 0)
    def _load_attractors():
        cp = pltpu.make_async_copy(att_hbm, att_vmem, sem)
        cp.start()
        cp.wait()
        KC = 512
        def _norm_step(i, _):
            blk = att_vmem[pl.ds(i * KC, KC), :]
            sq = jnp.sum(blk * blk, axis=1)
            inv_ref[0, pl.ds(i * KC, KC)] = (
                1.0 / jnp.maximum(jnp.sqrt(sq), EPS))
            return 0
        lax.fori_loop(0, att_vmem.shape[0] // KC, _norm_step, 0)

    Bb, S, D = hs_ref.shape
    SC = 32
    lane = lax.broadcasted_iota(jnp.int32, (Bb,), 0)

    acc0 = jnp.zeros((Bb, D), jnp.float32)
    acc1 = jnp.zeros((Bb, D), jnp.float32)
    for i in range(0, S // SC, 2):
        acc0 = acc0 + jnp.sum(hs_ref[:, i * SC:(i + 1) * SC, :], axis=1)
        acc1 = acc1 + jnp.sum(hs_ref[:, (i + 1) * SC:(i + 2) * SC, :], axis=1)
    v_raw = (acc0 + acc1) / S
    vn = jnp.sqrt(jnp.sum(v_raw * v_raw, axis=1, keepdims=True))
    v_unit = v_raw / jnp.maximum(vn, EPS)

    nch = logits_ref.shape[0] // Bb
    conf = jnp.zeros((Bb,), jnp.float32)
    for bi in range(Bb):
        x2 = logits_ref[bi * nch:(bi + 1) * nch, :]
        m_b = jnp.max(x2)
        xc = x2 - m_b
        ec = jnp.exp(xc)
        z_b = jnp.sum(ec)
        s1_b = jnp.sum(xc * ec)
        conf_b = 1.0 - (jnp.log(z_b) - s1_b / z_b) / logv
        conf = jnp.where(lane == bi, conf_b, conf)

    reg = reg_ref[...]
    rn = jnp.sqrt(jnp.sum(reg * reg, axis=1, keepdims=True))
    reg_n = reg / jnp.maximum(rn, EPS)
    rsims = lax.dot_general(v_unit, reg_n, (((1,), (1,)), ((), ())),
                            preferred_element_type=jnp.float32)
    iota_r = lax.broadcasted_iota(jnp.int32, rsims.shape, 1)
    rsims = jnp.where(iota_r < n_reg, rsims, -1e30)
    penalty = jnp.max(rsims, axis=1)

    K = att_vmem.shape[0]
    KB = 1024
    chunks = []
    for k in range(K // KB):
        att_c = att_vmem[k * KB:(k + 1) * KB, :]
        s_c = lax.dot_general(v_unit, att_c, (((1,), (1,)), ((), ())),
                              preferred_element_type=jnp.float32)
        chunks.append(s_c * inv_ref[0, k * KB:(k + 1) * KB][None, :])
    sims = jnp.concatenate(chunks, axis=1)
    best = jnp.max(sims, axis=1)
    idx = jnp.argmax(sims, axis=1).astype(jnp.int32)

    for bi in range(Bb):
        idx_i = jnp.sum(jnp.where(lane == bi, idx, 0))
        rows_ref[bi:bi + 1, :] = att_vmem[pl.ds(idx_i, 1), :]
    closest = rows_ref[...]
    cn = jnp.sqrt(jnp.sum(closest * closest, axis=1, keepdims=True))
    closest_n = closest / jnp.maximum(cn, EPS)

    alpha = (ALPHA_BASE * (1.0 - best)) * conf * (1.0 - penalty * 0.5)
    delta = jnp.clip(closest_n - v_unit, -MAX_DELTA, MAX_DELTA)
    steering = alpha[:, None] * delta

    def _add_step(i, _):
        out_ref[:, pl.ds(i * SC, SC), :] = (
            hs_ref[:, pl.ds(i * SC, SC), :] + steering[:, None, :])
        return 0
    lax.fori_loop(0, S // SC, _add_step, 0)


def kernel(hidden_states, logits, attractors, uncertainty_regions):
    B, S, D = hidden_states.shape
    K = attractors.shape[0]
    V = logits.shape[-1]
    R = uncertainty_regions.shape[0]
    logits_flat = logits[:, -1, :].reshape(B * V // 128, 128)
    Rp = max(8, ((R + 7) // 8) * 8)
    reg_p = jnp.pad(uncertainty_regions, ((0, Rp - R), (0, 0)))

    Bb = 8
    body = functools.partial(_fused_body, R, float(np.log(V)))
    return pl.pallas_call(
        body,
        grid=(B // Bb,),
        in_specs=[
            pl.BlockSpec((Bb, S, D), lambda b: (b, 0, 0)),
            pl.BlockSpec((Bb * V // 128, 128), lambda b: (b, 0)),
            pl.BlockSpec(memory_space=pl.ANY),
            pl.BlockSpec((Rp, D), lambda b: (0, 0)),
        ],
        out_specs=pl.BlockSpec((Bb, S, D), lambda b: (b, 0, 0)),
        out_shape=jax.ShapeDtypeStruct((B, S, D), jnp.float32),
        scratch_shapes=[
            pltpu.VMEM((K, D), jnp.float32),
            pltpu.VMEM((1, K), jnp.float32),
            pltpu.VMEM((Bb, D), jnp.float32),
            pltpu.SemaphoreType.DMA,
        ],
        compiler_params=pltpu.CompilerParams(
            vmem_limit_bytes=67108864),
    )(hidden_states, logits_flat, attractors, reg_p)
